# baseline (speedup 1.0000x reference)
import jax
import jax.numpy as jnp
from jax.experimental import pallas as pl
from jax.experimental.pallas import tpu as pltpu

_S = 4
_C = 16
_H = 32
_OC = 16
_CIN = _C + 4


def _agg_kernel(x_ref, wd_ref, bv_ref, wgb_ref, wvm_ref, bg_ref, wa_ref,
                ba_ref, wf_ref, bf_ref, out_ref):
    f32 = jnp.float32
    T = x_ref.shape[3]
    S, C, H, OC = _S, _C, _H, _OC

    featall = jnp.concatenate([x_ref[0, s, 0:C] for s in range(S)], axis=0)
    dall = jnp.concatenate([x_ref[0, s, C:_CIN] for s in range(S)], axis=0)

    vall = jnp.dot(wd_ref[...], dall, preferred_element_type=f32)
    vall = jnp.maximum(vall + jnp.broadcast_to(bv_ref[...], (S * C, T)), 0.0)
    img = featall + vall

    mean = (img[0:C] + img[C:2 * C] + img[2 * C:3 * C] + img[3 * C:4 * C]) * (1.0 / S)
    mean4 = jnp.concatenate([mean] * S, axis=0)
    dlt = img - mean4
    sq = dlt * dlt
    var = (sq[0:C] + sq[C:2 * C] + sq[2 * C:3 * C] + sq[3 * C:4 * C]) * (1.0 / (S - 1))

    vm = jnp.concatenate([var, mean], axis=0)
    gvm = (jnp.dot(wvm_ref[...], vm, preferred_element_type=f32)
           + jnp.broadcast_to(bg_ref[...], (H, T)))
    gvm4 = jnp.concatenate([gvm] * S, axis=0)
    gf = jnp.maximum(jnp.dot(wgb_ref[...], img, preferred_element_type=f32)
                     + gvm4, 0.0)

    p = gf * jnp.broadcast_to(wa_ref[...], (S * H, T))
    ba_b = jnp.broadcast_to(ba_ref[...], (1, T))
    scores = jnp.concatenate(
        [jnp.maximum(jnp.sum(p[H * s:H * s + H], axis=0, keepdims=True) + ba_b,
                     0.0) for s in range(S)], axis=0)

    m = jnp.max(scores, axis=0, keepdims=True)
    e = jnp.exp(scores - m)
    w = e * pl.reciprocal(jnp.sum(e, axis=0, keepdims=True), approx=False)
    acc = w[0:1] * gf[0:H]
    for s in range(1, S):
        acc = acc + w[s:s + 1] * gf[H * s:H * s + H]

    out = jnp.dot(wf_ref[...], acc, preferred_element_type=f32)
    out = out + jnp.broadcast_to(bf_ref[...], (OC, T))
    out_ref[...] = jnp.maximum(out, 0.0).reshape(1, OC, T).astype(out_ref.dtype)


def kernel(x, wv, bv, wg, bg, wa, ba, wf, bf, *, tile_n=1024):
    B, P, S, Cin = x.shape
    C = Cin - 4
    H = wg.shape[1]
    OC = wf.shape[1]
    N = B * P
    f32 = jnp.float32

    tile = next((t for t in (tile_n, 1024, 512, 256, 128) if P % t == 0), P)
    nb = N // tile
    jb = P // tile

    xt = jnp.transpose(x.reshape(nb, tile, S, Cin), (0, 2, 3, 1))

    wd = jnp.zeros((S * C, S * 4), f32)
    wgb = jnp.zeros((S * H, S * C), f32)
    for s in range(S):
        wd = wd.at[s * C:(s + 1) * C, s * 4:(s + 1) * 4].set(wv.T)
        wgb = wgb.at[s * H:(s + 1) * H, s * C:(s + 1) * C].set(wg[:C].T)
    bv4 = jnp.tile(bv.reshape(C, 1), (S, 1))
    wvm = jnp.concatenate([wg[C:2 * C].T, wg[2 * C:3 * C].T], axis=1)
    bg_c = bg.reshape(H, 1)
    wa4 = jnp.tile(wa.reshape(H, 1), (S, 1))
    ba_c = ba.reshape(1, 1)
    wfT = wf.T
    bf_c = bf.reshape(OC, 1)

    def full(a):
        nd = a.ndim
        return pl.BlockSpec(a.shape, lambda i, _nd=nd: (0,) * _nd)

    out = pl.pallas_call(
        _agg_kernel,
        out_shape=jax.ShapeDtypeStruct((B, OC, P), f32),
        grid=(nb,),
        in_specs=[
            pl.BlockSpec((1, S, Cin, tile), lambda i: (i, 0, 0, 0)),
            full(wd), full(bv4), full(wgb), full(wvm), full(bg_c),
            full(wa4), full(ba_c), full(wfT), full(bf_c),
        ],
        out_specs=pl.BlockSpec((1, OC, tile),
                               lambda i, _jb=jb: (i // _jb, 0, i % _jb)),
        compiler_params=pltpu.CompilerParams(
            dimension_semantics=("parallel",),
            vmem_limit_bytes=64 * 1024 * 1024),
    )(xt, wd, bv4, wgb, wvm, bg_c, wa4, ba_c, wfT, bf_c)

    return jnp.transpose(out, (0, 2, 1))


# speedup vs baseline: 1.4231x; 1.4231x over previous
import jax
import jax.numpy as jnp
from jax.experimental import pallas as pl
from jax.experimental.pallas import tpu as pltpu

_S = 4
_C = 16
_H = 32
_OC = 16
_CIN = _C + 4


def _agg_kernel(x_ref, wd_ref, bv_ref, wgb_ref, wvm_ref, bg_ref, wa_ref,
                ba_ref, wf_ref, bf_ref, out_ref):
    f32 = jnp.float32
    T = x_ref.shape[2]
    S, C, H, OC = _S, _C, _H, _OC

    featall = jnp.concatenate([x_ref[s, 0:C] for s in range(S)], axis=0)
    dall = jnp.concatenate([x_ref[s, C:_CIN] for s in range(S)], axis=0)

    vall = jnp.dot(wd_ref[...], dall, preferred_element_type=f32)
    vall = jnp.maximum(vall + jnp.broadcast_to(bv_ref[...], (S * C, T)), 0.0)
    img = featall + vall

    mean = (img[0:C] + img[C:2 * C] + img[2 * C:3 * C] + img[3 * C:4 * C]) * (1.0 / S)
    mean4 = jnp.concatenate([mean] * S, axis=0)
    dlt = img - mean4
    sq = dlt * dlt
    var = (sq[0:C] + sq[C:2 * C] + sq[2 * C:3 * C] + sq[3 * C:4 * C]) * (1.0 / (S - 1))

    vm = jnp.concatenate([var, mean], axis=0)
    gvm = (jnp.dot(wvm_ref[...], vm, preferred_element_type=f32)
           + jnp.broadcast_to(bg_ref[...], (H, T)))
    gvm4 = jnp.concatenate([gvm] * S, axis=0)
    gf = jnp.maximum(jnp.dot(wgb_ref[...], img, preferred_element_type=f32)
                     + gvm4, 0.0)

    p = gf * jnp.broadcast_to(wa_ref[...], (S * H, T))
    ba_b = jnp.broadcast_to(ba_ref[...], (1, T))
    scores = jnp.concatenate(
        [jnp.maximum(jnp.sum(p[H * s:H * s + H], axis=0, keepdims=True) + ba_b,
                     0.0) for s in range(S)], axis=0)

    m = jnp.max(scores, axis=0, keepdims=True)
    e = jnp.exp(scores - m)
    w = e * pl.reciprocal(jnp.sum(e, axis=0, keepdims=True), approx=False)
    acc = w[0:1] * gf[0:H]
    for s in range(1, S):
        acc = acc + w[s:s + 1] * gf[H * s:H * s + H]

    out = jnp.dot(wf_ref[...], acc, preferred_element_type=f32)
    out = out + jnp.broadcast_to(bf_ref[...], (OC, T))
    out_ref[...] = jnp.maximum(out, 0.0).reshape(1, OC, T).astype(out_ref.dtype)


def kernel(x, wv, bv, wg, bg, wa, ba, wf, bf, *, tile_n=2048):
    B, P, S, Cin = x.shape
    C = Cin - 4
    H = wg.shape[1]
    OC = wf.shape[1]
    N = B * P
    f32 = jnp.float32

    tile = next((t for t in (tile_n, 1024, 512, 256, 128) if P % t == 0), P)
    jb = P // tile

    xt = jnp.transpose(x.reshape(N, S, Cin), (1, 2, 0))

    wd = jnp.zeros((S * C, S * 4), f32)
    wgb = jnp.zeros((S * H, S * C), f32)
    for s in range(S):
        wd = wd.at[s * C:(s + 1) * C, s * 4:(s + 1) * 4].set(wv.T)
        wgb = wgb.at[s * H:(s + 1) * H, s * C:(s + 1) * C].set(wg[:C].T)
    bv4 = jnp.tile(bv.reshape(C, 1), (S, 1))
    wvm = jnp.concatenate([wg[C:2 * C].T, wg[2 * C:3 * C].T], axis=1)
    bg_c = bg.reshape(H, 1)
    wa4 = jnp.tile(wa.reshape(H, 1), (S, 1))
    ba_c = ba.reshape(1, 1)
    wfT = wf.T
    bf_c = bf.reshape(OC, 1)

    def full(a):
        nd = a.ndim
        return pl.BlockSpec(a.shape, lambda b, j, _nd=nd: (0,) * _nd)

    out = pl.pallas_call(
        _agg_kernel,
        out_shape=jax.ShapeDtypeStruct((B, OC, P), f32),
        grid=(B, jb),
        in_specs=[
            pl.BlockSpec((S, Cin, tile), lambda b, j, _jb=jb: (0, 0, b * _jb + j)),
            full(wd), full(bv4), full(wgb), full(wvm), full(bg_c),
            full(wa4), full(ba_c), full(wfT), full(bf_c),
        ],
        out_specs=pl.BlockSpec((1, OC, tile), lambda b, j: (b, 0, j)),
        compiler_params=pltpu.CompilerParams(
            dimension_semantics=("parallel", "parallel"),
            vmem_limit_bytes=64 * 1024 * 1024),
    )(xt, wd, bv4, wgb, wvm, bg_c, wa4, ba_c, wfT, bf_c)

    return jnp.transpose(out, (0, 2, 1))


# speedup vs baseline: 1.6151x; 1.1349x over previous
import jax
import jax.numpy as jnp
from jax.experimental import pallas as pl
from jax.experimental.pallas import tpu as pltpu

_S = 4
_C = 16
_H = 32
_OC = 16
_CIN = _C + 4


def _agg_kernel(x_ref, wd_ref, bv_ref, wgb_ref, wvm_ref, bg_ref, wa_ref,
                ba_ref, wf_ref, bf_ref, out_ref):
    f32 = jnp.float32
    T = x_ref.shape[2]
    S, C, H, OC = _S, _C, _H, _OC

    featall = jnp.concatenate([x_ref[s, 0:C] for s in range(S)], axis=0)
    dall = jnp.concatenate([x_ref[s, C:_CIN] for s in range(S)], axis=0)

    vall = jnp.dot(wd_ref[...], dall, preferred_element_type=f32)
    vall = jnp.maximum(vall + jnp.broadcast_to(bv_ref[...], (S * C, T)), 0.0)
    img = featall + vall

    mean = (img[0:C] + img[C:2 * C] + img[2 * C:3 * C] + img[3 * C:4 * C]) * (1.0 / S)
    mean4 = jnp.concatenate([mean] * S, axis=0)
    dlt = img - mean4
    sq = dlt * dlt
    var = (sq[0:C] + sq[C:2 * C] + sq[2 * C:3 * C] + sq[3 * C:4 * C]) * (1.0 / (S - 1))

    vm = jnp.concatenate([var, mean], axis=0)
    gvm = (jnp.dot(wvm_ref[...], vm, preferred_element_type=f32)
           + jnp.broadcast_to(bg_ref[...], (H, T)))
    gvm4 = jnp.concatenate([gvm] * S, axis=0)
    gf = jnp.maximum(jnp.dot(wgb_ref[...], img, preferred_element_type=f32)
                     + gvm4, 0.0)

    p = gf * jnp.broadcast_to(wa_ref[...], (S * H, T))
    ba_b = jnp.broadcast_to(ba_ref[...], (1, T))
    scores = jnp.concatenate(
        [jnp.maximum(jnp.sum(p[H * s:H * s + H], axis=0, keepdims=True) + ba_b,
                     0.0) for s in range(S)], axis=0)

    m = jnp.max(scores, axis=0, keepdims=True)
    e = jnp.exp(scores - m)
    w = e * pl.reciprocal(jnp.sum(e, axis=0, keepdims=True), approx=False)
    acc = w[0:1] * gf[0:H]
    for s in range(1, S):
        acc = acc + w[s:s + 1] * gf[H * s:H * s + H]

    out = jnp.dot(wf_ref[...], acc, preferred_element_type=f32)
    out = out + jnp.broadcast_to(bf_ref[...], (OC, T))
    out_ref[...] = jnp.maximum(out, 0.0).reshape(1, OC, T).astype(out_ref.dtype)


def kernel(x, wv, bv, wg, bg, wa, ba, wf, bf, *, tile_n=4096):
    B, P, S, Cin = x.shape
    C = Cin - 4
    H = wg.shape[1]
    OC = wf.shape[1]
    N = B * P
    f32 = jnp.float32

    tile = next((t for t in (tile_n, 1024, 512, 256, 128) if P % t == 0), P)
    jb = P // tile

    xt = jnp.transpose(x.reshape(N, S, Cin), (1, 2, 0))

    wd = jnp.zeros((S * C, S * 4), f32)
    wgb = jnp.zeros((S * H, S * C), f32)
    for s in range(S):
        wd = wd.at[s * C:(s + 1) * C, s * 4:(s + 1) * 4].set(wv.T)
        wgb = wgb.at[s * H:(s + 1) * H, s * C:(s + 1) * C].set(wg[:C].T)
    bv4 = jnp.tile(bv.reshape(C, 1), (S, 1))
    wvm = jnp.concatenate([wg[C:2 * C].T, wg[2 * C:3 * C].T], axis=1)
    bg_c = bg.reshape(H, 1)
    wa4 = jnp.tile(wa.reshape(H, 1), (S, 1))
    ba_c = ba.reshape(1, 1)
    wfT = wf.T
    bf_c = bf.reshape(OC, 1)

    def full(a):
        nd = a.ndim
        return pl.BlockSpec(a.shape, lambda b, j, _nd=nd: (0,) * _nd)

    out = pl.pallas_call(
        _agg_kernel,
        out_shape=jax.ShapeDtypeStruct((B, OC, P), f32),
        grid=(B, jb),
        in_specs=[
            pl.BlockSpec((S, Cin, tile), lambda b, j, _jb=jb: (0, 0, b * _jb + j)),
            full(wd), full(bv4), full(wgb), full(wvm), full(bg_c),
            full(wa4), full(ba_c), full(wfT), full(bf_c),
        ],
        out_specs=pl.BlockSpec((1, OC, tile), lambda b, j: (b, 0, j)),
        compiler_params=pltpu.CompilerParams(
            dimension_semantics=("parallel", "parallel"),
            vmem_limit_bytes=64 * 1024 * 1024),
    )(xt, wd, bv4, wgb, wvm, bg_c, wa4, ba_c, wfT, bf_c)

    return jnp.transpose(out, (0, 2, 1))


# speedup vs baseline: 1.7009x; 1.0532x over previous
import jax
import jax.numpy as jnp
from jax.experimental import pallas as pl
from jax.experimental.pallas import tpu as pltpu

_S = 4
_C = 16
_H = 32
_OC = 16
_CIN = _C + 4


def _agg_kernel(x_ref, wd_ref, bv_ref, wgb_ref, wvm_ref, bg_ref, wa_ref,
                ba_ref, wf_ref, bf_ref, out_ref):
    f32 = jnp.float32
    T = x_ref.shape[2]
    S, C, H, OC = _S, _C, _H, _OC

    featall = jnp.concatenate([x_ref[s, 0:C] for s in range(S)], axis=0)
    dall = jnp.concatenate([x_ref[s, C:_CIN] for s in range(S)], axis=0)

    vall = jnp.dot(wd_ref[...], dall, preferred_element_type=f32)
    vall = jnp.maximum(vall + jnp.broadcast_to(bv_ref[...], (S * C, T)), 0.0)
    img = featall + vall

    mean = (img[0:C] + img[C:2 * C] + img[2 * C:3 * C] + img[3 * C:4 * C]) * (1.0 / S)
    mean4 = jnp.concatenate([mean] * S, axis=0)
    dlt = img - mean4
    sq = dlt * dlt
    var = (sq[0:C] + sq[C:2 * C] + sq[2 * C:3 * C] + sq[3 * C:4 * C]) * (1.0 / (S - 1))

    vm = jnp.concatenate([var, mean], axis=0)
    gvm = (jnp.dot(wvm_ref[...], vm, preferred_element_type=f32)
           + jnp.broadcast_to(bg_ref[...], (H, T)))
    gvm4 = jnp.concatenate([gvm] * S, axis=0)
    gf = jnp.maximum(jnp.dot(wgb_ref[...], img, preferred_element_type=f32)
                     + gvm4, 0.0)

    p = gf * jnp.broadcast_to(wa_ref[...], (S * H, T))
    ba_b = jnp.broadcast_to(ba_ref[...], (1, T))
    scores = jnp.concatenate(
        [jnp.maximum(jnp.sum(p[H * s:H * s + H], axis=0, keepdims=True) + ba_b,
                     0.0) for s in range(S)], axis=0)

    m = jnp.max(scores, axis=0, keepdims=True)
    e = jnp.exp(scores - m)
    w = e * pl.reciprocal(jnp.sum(e, axis=0, keepdims=True), approx=False)
    acc = w[0:1] * gf[0:H]
    for s in range(1, S):
        acc = acc + w[s:s + 1] * gf[H * s:H * s + H]

    out = jnp.dot(wf_ref[...], acc, preferred_element_type=f32)
    out = out + jnp.broadcast_to(bf_ref[...], (OC, T))
    out_ref[...] = jnp.maximum(out, 0.0).reshape(1, OC, T).astype(out_ref.dtype)


def kernel(x, wv, bv, wg, bg, wa, ba, wf, bf, *, tile_n=8192):
    B, P, S, Cin = x.shape
    C = Cin - 4
    H = wg.shape[1]
    OC = wf.shape[1]
    N = B * P
    f32 = jnp.float32

    tile = next((t for t in (tile_n, 4096, 1024, 512, 256, 128) if P % t == 0), P)
    jb = P // tile

    xt = jnp.transpose(x.reshape(N, S, Cin), (1, 2, 0))

    wd = jnp.zeros((S * C, S * 4), f32)
    wgb = jnp.zeros((S * H, S * C), f32)
    for s in range(S):
        wd = wd.at[s * C:(s + 1) * C, s * 4:(s + 1) * 4].set(wv.T)
        wgb = wgb.at[s * H:(s + 1) * H, s * C:(s + 1) * C].set(wg[:C].T)
    bv4 = jnp.tile(bv.reshape(C, 1), (S, 1))
    wvm = jnp.concatenate([wg[C:2 * C].T, wg[2 * C:3 * C].T], axis=1)
    bg_c = bg.reshape(H, 1)
    wa4 = jnp.tile(wa.reshape(H, 1), (S, 1))
    ba_c = ba.reshape(1, 1)
    wfT = wf.T
    bf_c = bf.reshape(OC, 1)

    def full(a):
        nd = a.ndim
        return pl.BlockSpec(a.shape, lambda b, j, _nd=nd: (0,) * _nd)

    out = pl.pallas_call(
        _agg_kernel,
        out_shape=jax.ShapeDtypeStruct((B, OC, P), f32),
        grid=(B, jb),
        in_specs=[
            pl.BlockSpec((S, Cin, tile), lambda b, j, _jb=jb: (0, 0, b * _jb + j)),
            full(wd), full(bv4), full(wgb), full(wvm), full(bg_c),
            full(wa4), full(ba_c), full(wfT), full(bf_c),
        ],
        out_specs=pl.BlockSpec((1, OC, tile), lambda b, j: (b, 0, j)),
        compiler_params=pltpu.CompilerParams(
            dimension_semantics=("parallel", "parallel"),
            vmem_limit_bytes=64 * 1024 * 1024),
    )(xt, wd, bv4, wgb, wvm, bg_c, wa4, ba_c, wfT, bf_c)

    return jnp.transpose(out, (0, 2, 1))


# speedup vs baseline: 1.7124x; 1.0067x over previous
import jax
import jax.numpy as jnp
from jax.experimental import pallas as pl
from jax.experimental.pallas import tpu as pltpu

_S = 4
_C = 16
_H = 32
_OC = 16
_CIN = _C + 4


def _agg_kernel(x_ref, wd_ref, bv_ref, wgb_ref, wvm_ref, bg_ref, wa_ref,
                ba_ref, wf_ref, bf_ref, out_ref):
    f32 = jnp.float32
    T = x_ref.shape[2]
    S, C, H, OC = _S, _C, _H, _OC

    featall = jnp.concatenate([x_ref[s, 0:C] for s in range(S)], axis=0)
    dall = jnp.concatenate([x_ref[s, C:_CIN] for s in range(S)], axis=0)

    vall = jnp.dot(wd_ref[...], dall, preferred_element_type=f32)
    vall = jnp.maximum(vall + jnp.broadcast_to(bv_ref[...], (S * C, T)), 0.0)
    img = featall + vall

    mean = (img[0:C] + img[C:2 * C] + img[2 * C:3 * C] + img[3 * C:4 * C]) * (1.0 / S)
    mean4 = jnp.concatenate([mean] * S, axis=0)
    dlt = img - mean4
    sq = dlt * dlt
    var = (sq[0:C] + sq[C:2 * C] + sq[2 * C:3 * C] + sq[3 * C:4 * C]) * (1.0 / (S - 1))

    vm = jnp.concatenate([var, mean], axis=0)
    gvm = (jnp.dot(wvm_ref[...], vm, preferred_element_type=f32)
           + jnp.broadcast_to(bg_ref[...], (H, T)))
    gvm4 = jnp.concatenate([gvm] * S, axis=0)
    gf = jnp.maximum(jnp.dot(wgb_ref[...], img, preferred_element_type=f32)
                     + gvm4, 0.0)

    p = gf * jnp.broadcast_to(wa_ref[...], (S * H, T))
    ba_b = jnp.broadcast_to(ba_ref[...], (1, T))
    scores = jnp.concatenate(
        [jnp.maximum(jnp.sum(p[H * s:H * s + H], axis=0, keepdims=True) + ba_b,
                     0.0) for s in range(S)], axis=0)

    m = jnp.max(scores, axis=0, keepdims=True)
    e = jnp.exp(scores - m)
    w = e * pl.reciprocal(jnp.sum(e, axis=0, keepdims=True), approx=False)
    acc = w[0:1] * gf[0:H]
    for s in range(1, S):
        acc = acc + w[s:s + 1] * gf[H * s:H * s + H]

    out = jnp.dot(wf_ref[...], acc, preferred_element_type=f32)
    out = out + jnp.broadcast_to(bf_ref[...], (OC, T))
    out_ref[...] = jnp.maximum(out, 0.0).reshape(1, OC, T).astype(out_ref.dtype)


def kernel(x, wv, bv, wg, bg, wa, ba, wf, bf, *, tile_n=16384):
    B, P, S, Cin = x.shape
    C = Cin - 4
    H = wg.shape[1]
    OC = wf.shape[1]
    N = B * P
    f32 = jnp.float32

    tile = next((t for t in (tile_n, 8192, 4096, 1024, 512, 256, 128) if P % t == 0), P)
    jb = P // tile

    xt = jnp.transpose(x.reshape(N, S, Cin), (1, 2, 0))

    wd = jnp.zeros((S * C, S * 4), f32)
    wgb = jnp.zeros((S * H, S * C), f32)
    for s in range(S):
        wd = wd.at[s * C:(s + 1) * C, s * 4:(s + 1) * 4].set(wv.T)
        wgb = wgb.at[s * H:(s + 1) * H, s * C:(s + 1) * C].set(wg[:C].T)
    bv4 = jnp.tile(bv.reshape(C, 1), (S, 1))
    wvm = jnp.concatenate([wg[C:2 * C].T, wg[2 * C:3 * C].T], axis=1)
    bg_c = bg.reshape(H, 1)
    wa4 = jnp.tile(wa.reshape(H, 1), (S, 1))
    ba_c = ba.reshape(1, 1)
    wfT = wf.T
    bf_c = bf.reshape(OC, 1)

    def full(a):
        nd = a.ndim
        return pl.BlockSpec(a.shape, lambda b, j, _nd=nd: (0,) * _nd)

    out = pl.pallas_call(
        _agg_kernel,
        out_shape=jax.ShapeDtypeStruct((B, OC, P), f32),
        grid=(B, jb),
        in_specs=[
            pl.BlockSpec((S, Cin, tile), lambda b, j, _jb=jb: (0, 0, b * _jb + j)),
            full(wd), full(bv4), full(wgb), full(wvm), full(bg_c),
            full(wa4), full(ba_c), full(wfT), full(bf_c),
        ],
        out_specs=pl.BlockSpec((1, OC, tile), lambda b, j: (b, 0, j)),
        compiler_params=pltpu.CompilerParams(
            dimension_semantics=("parallel", "parallel"),
            vmem_limit_bytes=64 * 1024 * 1024),
    )(xt, wd, bv4, wgb, wvm, bg_c, wa4, ba_c, wfT, bf_c)

    return jnp.transpose(out, (0, 2, 1))
